# baseline (speedup 1.0000x reference)
import functools

import jax
import jax.numpy as jnp
from jax import lax
from jax.experimental import pallas as pl
from jax.experimental.pallas import tpu as pltpu
from jax.experimental.pallas import tpu_sc as plsc

B, T, N, F = 4, 24, 1000, 8
D = 128
E = 3000
SNAP = B * T
NPAD = 1024
EPAD = 3072
WCOL = SNAP * F
EPT = EPAD // 32
RPT = NPAD // 16


def _prep_body(x_ref, g_ref, b_ref, o_ref):
    xc = x_ref[...]
    gi = lax.broadcasted_iota(jnp.int32, (16 * F, 16 * F), 0) // F
    gj = lax.broadcasted_iota(jnp.int32, (16 * F, 16 * F), 1) // F
    gm = jnp.where(gi == gj, 1.0 / F, 0.0)
    mu = jnp.dot(xc, gm, preferred_element_type=jnp.float32)
    s2 = jnp.dot(xc * xc, gm, preferred_element_type=jnp.float32)
    var = s2 - mu * mu
    o_ref[...] = ((xc - mu) * lax.rsqrt(var + 1e-5) * g_ref[...]
                  + b_ref[...])


def _prep(xd, gtile, btile):
    rows = SNAP * N * F // (16 * F)
    return pl.pallas_call(
        _prep_body,
        in_specs=[
            pl.BlockSpec((rows, 16 * F), lambda: (0, 0)),
            pl.BlockSpec((1, 16 * F), lambda: (0, 0)),
            pl.BlockSpec((1, 16 * F), lambda: (0, 0)),
        ],
        out_specs=pl.BlockSpec((rows, 16 * F), lambda: (0, 0)),
        out_shape=jax.ShapeDtypeStruct((rows, 16 * F), jnp.float32),
    )(xd, gtile, btile)


def _stats_body(src_ref, dst_ref, s_ref, w_ref):
    iii = lax.broadcasted_iota(jnp.int32, (NPAD, 1), 0)
    iif = iii.astype(jnp.float32)
    ost = (iif == src_ref[...]).astype(jnp.bfloat16)
    odt = (iif == dst_ref[...]).astype(jnp.bfloat16)
    at = lax.dot_general(ost, odt, (((1,), (1,)), ((), ())),
                         preferred_element_type=jnp.float32)
    ones_col = jnp.ones((NPAD, 1), jnp.float32)
    deg = lax.dot_general(at, ones_col, (((0,), (0,)), ((), ())),
                          preferred_element_type=jnp.float32)
    s = 1.0 / (deg + 1.0)
    wext = jnp.dot(at, s, preferred_element_type=jnp.float32)
    w = jnp.where(iii < N, s + wext, 0.0)
    s_ref[...] = s
    w_ref[...] = w


def _graph_stats(srcf, dstf):
    return pl.pallas_call(
        _stats_body,
        in_specs=[
            pl.BlockSpec((1, EPAD), lambda: (0, 0)),
            pl.BlockSpec((1, EPAD), lambda: (0, 0)),
        ],
        out_specs=[
            pl.BlockSpec((NPAD, 1), lambda: (0, 0)),
            pl.BlockSpec((NPAD, 1), lambda: (0, 0)),
        ],
        out_shape=[
            jax.ShapeDtypeStruct((NPAD, 1), jnp.float32),
            jax.ShapeDtypeStruct((NPAD, 1), jnp.float32),
        ],
    )(srcf, dstf)


def _sc_edge_body(x_hbm, src_hbm, dst_hbm, out_hbm,
                  idx_s, idx_d, rows, sem, sem2):
    c = lax.axis_index("c")
    sc = lax.axis_index("s")
    base = (c * 16 + sc) * EPT
    zv = jnp.zeros((16,), jnp.float32)

    def _zrow(r, _):
        for j in range(WCOL // 16):
            rows[r, pl.ds(j * 16, 16)] = zv
        return 0

    lax.fori_loop(0, RPT, _zrow, 0)
    pltpu.sync_copy(rows.at[pl.ds(0, RPT)],
                    out_hbm.at[c, pl.ds(sc * RPT, RPT)])
    pltpu.sync_copy(src_hbm.at[pl.ds(base, EPT)], idx_s)
    pltpu.sync_copy(dst_hbm.at[pl.ds(base, EPT)], idx_d)
    pltpu.async_copy(x_hbm.at[idx_s], rows, sem).wait()
    plsc.subcore_barrier()
    pltpu.async_copy(rows, out_hbm.at[c].at[idx_d], sem2, add=True).wait()


def _sc_edge_agg(x_all, src_i, dst_i):
    mesh = plsc.VectorSubcoreMesh(core_axis_name="c", subcore_axis_name="s")
    fn = functools.partial(
        pl.kernel,
        out_type=jax.ShapeDtypeStruct((2, NPAD, WCOL), jnp.float32),
        mesh=mesh,
        scratch_types=[
            pltpu.VMEM((EPT,), jnp.int32),
            pltpu.VMEM((EPT,), jnp.int32),
            pltpu.VMEM((EPT, WCOL), jnp.float32),
            pltpu.SemaphoreType.DMA,
            pltpu.SemaphoreType.DMA,
        ],
    )(_sc_edge_body)
    return fn(x_all, src_i, dst_i)


def _nodemlp_body(x_ref, m_ref, s_ref, w_ref, w1_ref, b1_ref, o_ref):
    agg = (x_ref[...] + m_ref[0] + m_ref[1]) * s_ref[...]
    h = jnp.dot(agg, w1_ref[...], preferred_element_type=jnp.float32)
    h = jnp.maximum(h + b1_ref[...], 0.0)
    p = jnp.sum(w_ref[...] * h, axis=0, keepdims=True)
    o_ref[...] = p.reshape(1, 1, 16 * D)


def _node_mlp(x_all, m2, s, w, w1big, b1big):
    nq = WCOL // D
    return pl.pallas_call(
        _nodemlp_body,
        grid=(nq,),
        in_specs=[
            pl.BlockSpec((NPAD, D), lambda q: (0, q)),
            pl.BlockSpec((2, NPAD, D), lambda q: (0, 0, q)),
            pl.BlockSpec((NPAD, 1), lambda q: (0, 0)),
            pl.BlockSpec((NPAD, 1), lambda q: (0, 0)),
            pl.BlockSpec((D, 16 * D), lambda q: (0, 0)),
            pl.BlockSpec((1, 16 * D), lambda q: (0, 0)),
        ],
        out_specs=pl.BlockSpec((1, 1, 16 * D), lambda q: (q, 0, 0)),
        out_shape=jax.ShapeDtypeStruct((nq, 1, 16 * D), jnp.float32),
    )(x_all, m2, s, w, w1big, b1big)


def _rnn_body(p_ref, w2_ref, b2_ref,
              wih0_ref, whh0_ref, b0_ref,
              w11_ref, b1r_ref,
              btab_ref, bidx_ref,
              wh1_ref, bh1_ref, wh2_ref, bh2_ref, wh3_ref, bh3_ref,
              o_ref):
    whh0 = whh0_ref[...]
    b0 = b0_ref[...]
    w11 = w11_ref[...]
    b1r = b1r_ref[...]

    emb = (jnp.dot(p_ref[...] * (1.0 / N), w2_ref[...],
                   preferred_element_type=jnp.float32) + b2_ref[...])
    x1 = jnp.dot(emb, wih0_ref[...],
                 preferred_element_type=jnp.float32)

    def gates(g, c):
        i = jax.nn.sigmoid(g[:, 0:D])
        f = jax.nn.sigmoid(g[:, D:2 * D])
        gg = jnp.tanh(g[:, 2 * D:3 * D])
        o = jax.nn.sigmoid(g[:, 3 * D:4 * D])
        cn = f * c + i * gg
        return o * jnp.tanh(cn), cn

    z = jnp.zeros((B, D), jnp.float32)
    h1, c1, h2, c2 = z, z, z, z
    for t in range(T):
        g1 = (x1[4 * t:4 * t + 4]
              + jnp.dot(h1, whh0, preferred_element_type=jnp.float32) + b0)
        h1, c1 = gates(g1, c1)
        hcat = jnp.concatenate([h1, h2], axis=1)
        g2 = jnp.dot(hcat, w11, preferred_element_type=jnp.float32) + b1r
        h2, c2 = gates(g2, c2)

    idx = bidx_ref[0, 0]
    sel = (lax.broadcasted_iota(jnp.int32, (4, 1), 0) == idx)
    bemb = jnp.sum(btab_ref[...] * sel.astype(jnp.float32),
                   axis=0, keepdims=True)
    comb = h2 + bemb
    a1 = jnp.dot(comb, wh1_ref[...], preferred_element_type=jnp.float32)
    a1 = a1 + bh1_ref[...]
    a1 = a1 * jax.nn.sigmoid(a1)
    a2 = jnp.dot(a1, wh2_ref[...], preferred_element_type=jnp.float32)
    a2 = a2 + bh2_ref[...]
    a2 = a2 * jax.nn.sigmoid(a2)
    pred = jnp.dot(a2, wh3_ref[...], preferred_element_type=jnp.float32)
    pred = pred + bh3_ref[...]
    sp = jnp.log(1.0 + jnp.exp(-jnp.abs(pred))) + jnp.maximum(pred, 0.0)
    o_ref[...] = sp + 1e-6


def _rnn_head(pflat, w2, b2, wih0t, whh0t, b0, w11, b1r,
              btab, bidx, wh1, bh1, wh2, bh2, wh3p, bh3p):
    full = lambda shape: pl.BlockSpec(shape, lambda: tuple(0 for _ in shape))
    return pl.pallas_call(
        _rnn_body,
        in_specs=[
            full((SNAP, D)),
            full((D, D)), full((1, D)),
            full((D, 4 * D)), full((D, 4 * D)), full((1, 4 * D)),
            full((2 * D, 4 * D)), full((1, 4 * D)),
            full((4, D)),
            pl.BlockSpec(memory_space=pltpu.SMEM),
            full((D, D)), full((1, D)),
            full((D, 64)), full((1, 64)),
            full((64, D)), full((1, D)),
        ],
        out_specs=full((B, D)),
        out_shape=jax.ShapeDtypeStruct((B, D), jnp.float32),
    )(pflat, w2, b2, wih0t, whh0t, b0, w11, b1r,
      btab, bidx, wh1, bh1, wh2, bh2, wh3p, bh3p)


def kernel(snapshot_sequence, edge_index, branch_idx, gamma, beta,
           W1, b1, W2, b2, Wih0, Whh0, bih0, bhh0, Wih1, Whh1, bih1, bhh1,
           branch_table, Wh1, bh1, Wh2, bh2, Wh3, bh3):
    f32 = jnp.float32

    gtile = jnp.tile(gamma, 16).reshape(1, 16 * F)
    btile = jnp.tile(beta, 16).reshape(1, 16 * F)
    xd = snapshot_sequence.reshape(SNAP * N * F // (16 * F), 16 * F)
    y = _prep(xd, gtile, btile)
    y4 = y.reshape(B, T, N, F)
    x_all = jnp.transpose(y4, (2, 1, 0, 3)).reshape(N, WCOL)
    x_all = jnp.pad(x_all, ((0, NPAD - N), (0, 0)))

    src_i = jnp.concatenate(
        [edge_index[0], jnp.full((EPAD - E,), NPAD - 1, jnp.int32)])
    dst_i = jnp.concatenate(
        [edge_index[1], jnp.full((EPAD - E,), NPAD - 1, jnp.int32)])

    s, w = _graph_stats(src_i.astype(f32).reshape(1, EPAD),
                        dst_i.astype(f32).reshape(1, EPAD))

    m2 = _sc_edge_agg(x_all, src_i, dst_i)

    w1big = jnp.kron(jnp.eye(16, dtype=f32), W1)
    b1big = jnp.tile(b1, 16).reshape(1, 16 * D)
    pst = _node_mlp(x_all, m2, s, w, w1big, b1big)
    pflat = pst.reshape(SNAP, D)

    bidx = jnp.asarray(branch_idx, jnp.int32).reshape(1, 1)
    wh3p = jnp.pad(Wh3, ((0, 0), (0, D - 2)))
    bh3p = jnp.pad(bh3, (0, D - 2)).reshape(1, D)
    w11 = jnp.concatenate([Wih1.T, Whh1.T], axis=0)
    out = _rnn_head(
        pflat, W2, b2.reshape(1, D),
        Wih0.T, Whh0.T, (bih0 + bhh0).reshape(1, 4 * D),
        w11, (bih1 + bhh1).reshape(1, 4 * D),
        branch_table, bidx,
        Wh1, bh1.reshape(1, D), Wh2, bh2.reshape(1, 64),
        wh3p, bh3p)
    return out[:, :2]


# speedup vs baseline: 1.7873x; 1.7873x over previous
import functools

import jax
import jax.numpy as jnp
from jax import lax
from jax.experimental import pallas as pl
from jax.experimental.pallas import tpu as pltpu
from jax.experimental.pallas import tpu_sc as plsc

B, T, N, F = 4, 24, 1000, 8
D = 128
E = 3000
SNAP = B * T
NPAD = 1024
EPAD = 3072
WCOL = SNAP * F
EPT = EPAD // 32
RPT = NPAD // 16


def _prep_body(x_ref, g_ref, b_ref, o_ref):
    val = x_ref[...]
    parts = []
    for tl in range(4):
        for b in range(4):
            parts.append(val[b, tl])
    xc = jnp.concatenate(parts, axis=1)
    xc = jnp.concatenate([xc, jnp.zeros((NPAD - N, 16 * F), jnp.float32)],
                         axis=0)
    gi = lax.broadcasted_iota(jnp.int32, (16 * F, 16 * F), 0) // F
    gj = lax.broadcasted_iota(jnp.int32, (16 * F, 16 * F), 1) // F
    gm = jnp.where(gi == gj, 1.0 / F, 0.0)
    mu = jnp.dot(xc, gm, preferred_element_type=jnp.float32)
    s2 = jnp.dot(xc * xc, gm, preferred_element_type=jnp.float32)
    var = s2 - mu * mu
    o_ref[...] = ((xc - mu) * lax.rsqrt(var + 1e-5) * g_ref[...]
                  + b_ref[...])


def _prep(x4, gtile, btile):
    return pl.pallas_call(
        _prep_body,
        grid=(WCOL // (16 * F),),
        in_specs=[
            pl.BlockSpec((B, 4, N, F), lambda q: (0, q, 0, 0)),
            pl.BlockSpec((1, 16 * F), lambda q: (0, 0)),
            pl.BlockSpec((1, 16 * F), lambda q: (0, 0)),
        ],
        out_specs=pl.BlockSpec((NPAD, 16 * F), lambda q: (0, q)),
        out_shape=jax.ShapeDtypeStruct((NPAD, WCOL), jnp.float32),
    )(x4, gtile, btile)


def _stats_body(src_ref, dst_ref, s_ref, w_ref):
    iii = lax.broadcasted_iota(jnp.int32, (NPAD, 1), 0)
    iif = iii.astype(jnp.float32)
    ost = (iif == src_ref[...]).astype(jnp.bfloat16)
    odt = (iif == dst_ref[...]).astype(jnp.bfloat16)
    at = lax.dot_general(ost, odt, (((1,), (1,)), ((), ())),
                         preferred_element_type=jnp.float32)
    ones_col = jnp.ones((NPAD, 1), jnp.float32)
    deg = lax.dot_general(at, ones_col, (((0,), (0,)), ((), ())),
                          preferred_element_type=jnp.float32)
    s = 1.0 / (deg + 1.0)
    wext = jnp.dot(at, s, preferred_element_type=jnp.float32)
    w = jnp.where(iii < N, s + wext, 0.0)
    s_ref[...] = s
    w_ref[...] = w


def _graph_stats(srcf, dstf):
    return pl.pallas_call(
        _stats_body,
        in_specs=[
            pl.BlockSpec((1, EPAD), lambda: (0, 0)),
            pl.BlockSpec((1, EPAD), lambda: (0, 0)),
        ],
        out_specs=[
            pl.BlockSpec((NPAD, 1), lambda: (0, 0)),
            pl.BlockSpec((NPAD, 1), lambda: (0, 0)),
        ],
        out_shape=[
            jax.ShapeDtypeStruct((NPAD, 1), jnp.float32),
            jax.ShapeDtypeStruct((NPAD, 1), jnp.float32),
        ],
    )(srcf, dstf)


def _sc_edge_body(x_hbm, src_hbm, dst_hbm, out_hbm,
                  idx_s, idx_d, rows, sem, sem2):
    c = lax.axis_index("c")
    sc = lax.axis_index("s")
    base = (c * 16 + sc) * EPT
    zv = jnp.zeros((16,), jnp.float32)

    def _zrow(r, _):
        for j in range(WCOL // 16):
            rows[r, pl.ds(j * 16, 16)] = zv
        return 0

    lax.fori_loop(0, RPT, _zrow, 0)
    pltpu.sync_copy(rows.at[pl.ds(0, RPT)],
                    out_hbm.at[c, pl.ds(sc * RPT, RPT)])
    pltpu.sync_copy(src_hbm.at[pl.ds(base, EPT)], idx_s)
    pltpu.sync_copy(dst_hbm.at[pl.ds(base, EPT)], idx_d)
    pltpu.async_copy(x_hbm.at[idx_s], rows, sem).wait()
    plsc.subcore_barrier()
    pltpu.async_copy(rows, out_hbm.at[c].at[idx_d], sem2, add=True).wait()


def _sc_edge_agg(x_all, src_i, dst_i):
    mesh = plsc.VectorSubcoreMesh(core_axis_name="c", subcore_axis_name="s")
    fn = functools.partial(
        pl.kernel,
        out_type=jax.ShapeDtypeStruct((2, NPAD, WCOL), jnp.float32),
        mesh=mesh,
        scratch_types=[
            pltpu.VMEM((EPT,), jnp.int32),
            pltpu.VMEM((EPT,), jnp.int32),
            pltpu.VMEM((EPT, WCOL), jnp.float32),
            pltpu.SemaphoreType.DMA,
            pltpu.SemaphoreType.DMA,
        ],
    )(_sc_edge_body)
    return fn(x_all, src_i, dst_i)


def _nodemlp_body(x_ref, m_ref, s_ref, w_ref, w1_ref, b1_ref, o_ref):
    agg = (x_ref[...] + m_ref[0] + m_ref[1]) * s_ref[...]
    h = jnp.dot(agg, w1_ref[...], preferred_element_type=jnp.float32)
    h = jnp.maximum(h + b1_ref[...], 0.0)
    p = jnp.sum(w_ref[...] * h, axis=0, keepdims=True)
    o_ref[...] = p.reshape(1, 1, 16 * D)


def _node_mlp(x_all, m2, s, w, w1big, b1big):
    nq = WCOL // D
    return pl.pallas_call(
        _nodemlp_body,
        grid=(nq,),
        in_specs=[
            pl.BlockSpec((NPAD, D), lambda q: (0, q)),
            pl.BlockSpec((2, NPAD, D), lambda q: (0, 0, q)),
            pl.BlockSpec((NPAD, 1), lambda q: (0, 0)),
            pl.BlockSpec((NPAD, 1), lambda q: (0, 0)),
            pl.BlockSpec((D, 16 * D), lambda q: (0, 0)),
            pl.BlockSpec((1, 16 * D), lambda q: (0, 0)),
        ],
        out_specs=pl.BlockSpec((1, 1, 16 * D), lambda q: (q, 0, 0)),
        out_shape=jax.ShapeDtypeStruct((nq, 1, 16 * D), jnp.float32),
    )(x_all, m2, s, w, w1big, b1big)


def _rnn_body(p_ref, w2_ref, b2_ref,
              wih0_ref, whh0_ref, b0_ref,
              w11_ref, b1r_ref,
              btab_ref, bidx_ref,
              wh1_ref, bh1_ref, wh2_ref, bh2_ref, wh3_ref, bh3_ref,
              o_ref):
    whh0 = whh0_ref[...]
    b0 = b0_ref[...]
    w11 = w11_ref[...]
    b1r = b1r_ref[...]

    emb = (jnp.dot(p_ref[...] * (1.0 / N), w2_ref[...],
                   preferred_element_type=jnp.float32) + b2_ref[...])
    x1 = jnp.dot(emb, wih0_ref[...],
                 preferred_element_type=jnp.float32)

    def gates(g, c):
        i = jax.nn.sigmoid(g[:, 0:D])
        f = jax.nn.sigmoid(g[:, D:2 * D])
        gg = jnp.tanh(g[:, 2 * D:3 * D])
        o = jax.nn.sigmoid(g[:, 3 * D:4 * D])
        cn = f * c + i * gg
        return o * jnp.tanh(cn), cn

    z = jnp.zeros((B, D), jnp.float32)
    h1, c1, h2, c2 = z, z, z, z
    for t in range(T):
        g1 = (x1[4 * t:4 * t + 4]
              + jnp.dot(h1, whh0, preferred_element_type=jnp.float32) + b0)
        h1, c1 = gates(g1, c1)
        hcat = jnp.concatenate([h1, h2], axis=1)
        g2 = jnp.dot(hcat, w11, preferred_element_type=jnp.float32) + b1r
        h2, c2 = gates(g2, c2)

    idx = bidx_ref[0, 0]
    sel = (lax.broadcasted_iota(jnp.int32, (4, 1), 0) == idx)
    bemb = jnp.sum(btab_ref[...] * sel.astype(jnp.float32),
                   axis=0, keepdims=True)
    comb = h2 + bemb
    a1 = jnp.dot(comb, wh1_ref[...], preferred_element_type=jnp.float32)
    a1 = a1 + bh1_ref[...]
    a1 = a1 * jax.nn.sigmoid(a1)
    a2 = jnp.dot(a1, wh2_ref[...], preferred_element_type=jnp.float32)
    a2 = a2 + bh2_ref[...]
    a2 = a2 * jax.nn.sigmoid(a2)
    pred = jnp.dot(a2, wh3_ref[...], preferred_element_type=jnp.float32)
    pred = pred + bh3_ref[...]
    sp = jnp.log(1.0 + jnp.exp(-jnp.abs(pred))) + jnp.maximum(pred, 0.0)
    o_ref[...] = sp + 1e-6


def _rnn_head(pflat, w2, b2, wih0t, whh0t, b0, w11, b1r,
              btab, bidx, wh1, bh1, wh2, bh2, wh3p, bh3p):
    full = lambda shape: pl.BlockSpec(shape, lambda: tuple(0 for _ in shape))
    return pl.pallas_call(
        _rnn_body,
        in_specs=[
            full((SNAP, D)),
            full((D, D)), full((1, D)),
            full((D, 4 * D)), full((D, 4 * D)), full((1, 4 * D)),
            full((2 * D, 4 * D)), full((1, 4 * D)),
            full((4, D)),
            pl.BlockSpec(memory_space=pltpu.SMEM),
            full((D, D)), full((1, D)),
            full((D, 64)), full((1, 64)),
            full((64, D)), full((1, D)),
        ],
        out_specs=full((B, D)),
        out_shape=jax.ShapeDtypeStruct((B, D), jnp.float32),
    )(pflat, w2, b2, wih0t, whh0t, b0, w11, b1r,
      btab, bidx, wh1, bh1, wh2, bh2, wh3p, bh3p)


def kernel(snapshot_sequence, edge_index, branch_idx, gamma, beta,
           W1, b1, W2, b2, Wih0, Whh0, bih0, bhh0, Wih1, Whh1, bih1, bhh1,
           branch_table, Wh1, bh1, Wh2, bh2, Wh3, bh3):
    f32 = jnp.float32

    gtile = jnp.tile(gamma, 16).reshape(1, 16 * F)
    btile = jnp.tile(beta, 16).reshape(1, 16 * F)
    x_all = _prep(snapshot_sequence, gtile, btile)

    src_i = jnp.concatenate(
        [edge_index[0], jnp.full((EPAD - E,), NPAD - 1, jnp.int32)])
    dst_i = jnp.concatenate(
        [edge_index[1], jnp.full((EPAD - E,), NPAD - 1, jnp.int32)])

    s, w = _graph_stats(src_i.astype(f32).reshape(1, EPAD),
                        dst_i.astype(f32).reshape(1, EPAD))

    m2 = _sc_edge_agg(x_all, src_i, dst_i)

    w1big = jnp.kron(jnp.eye(16, dtype=f32), W1)
    b1big = jnp.tile(b1, 16).reshape(1, 16 * D)
    pst = _node_mlp(x_all, m2, s, w, w1big, b1big)
    pflat = pst.reshape(SNAP, D)

    bidx = jnp.asarray(branch_idx, jnp.int32).reshape(1, 1)
    wh3p = jnp.pad(Wh3, ((0, 0), (0, D - 2)))
    bh3p = jnp.pad(bh3, (0, D - 2)).reshape(1, D)
    w11 = jnp.concatenate([Wih1.T, Whh1.T], axis=0)
    out = _rnn_head(
        pflat, W2, b2.reshape(1, D),
        Wih0.T, Whh0.T, (bih0 + bhh0).reshape(1, 4 * D),
        w11, (bih1 + bhh1).reshape(1, 4 * D),
        branch_table, bidx,
        Wh1, bh1.reshape(1, D), Wh2, bh2.reshape(1, 64),
        wh3p, bh3p)
    return out[:, :2]
